# Initial kernel scaffold; baseline (speedup 1.0000x reference)
import jax
import jax.numpy as jnp
from jax.experimental import pallas as pl


def kernel(x, edge_index, W1, b1, W2, b2, W_ntg, b_ntg, W_t, b_t):
    raise NotImplementedError("write your pallas kernel here")


# speedup vs baseline: 3.8853x; 3.8853x over previous
import functools

import jax
import jax.numpy as jnp
from jax import lax
from jax.experimental import pallas as pl
from jax.experimental.pallas import tpu as pltpu
from jax.experimental.pallas import tpu_sc as plsc

MAX_DEG = 10
NC, NS = 2, 16
NW = NC * NS
CHUNK = 128


def _make_segsum(n_table, d_w, n_rows, n_chunks_per_worker):
    assert d_w % 16 == 0 and n_rows % (NS * 8) == 0
    rows_per_tile = n_rows // NS
    mesh = plsc.VectorSubcoreMesh(core_axis_name="c", subcore_axis_name="s")

    @functools.partial(
        pl.kernel,
        mesh=mesh,
        out_type=jax.ShapeDtypeStruct((NC, n_rows, d_w), jnp.float32),
        scratch_types=[
            pltpu.VMEM((CHUNK,), jnp.int32),
            pltpu.VMEM((CHUNK,), jnp.int32),
            pltpu.VMEM((CHUNK, d_w), jnp.float32),
            pltpu.VMEM_SHARED((n_rows, d_w), jnp.float32),
            pltpu.SemaphoreType.DMA,
        ],
        compiler_params=pltpu.CompilerParams(use_tc_tiling_on_sc=False),
    )
    def seg_kernel(table_hbm, src_hbm, dst_hbm, out_hbm,
                   src_v, dst_v, rows_v, acc_sh, sem):
        c = lax.axis_index("c")
        s = lax.axis_index("s")
        wid = s * NC + c

        def _zrow(r, _):
            for cc in range(d_w // 16):
                rows_v[r, pl.ds(cc * 16, 16)] = jnp.zeros((16,), jnp.float32)
            return 0
        lax.fori_loop(0, CHUNK, _zrow, 0)

        tile_base = pl.multiple_of(s * rows_per_tile, 8)
        done = 0
        while done < rows_per_tile:
            nr = min(CHUNK, rows_per_tile - done)
            pltpu.sync_copy(rows_v.at[pl.ds(0, nr)],
                            acc_sh.at[pl.ds(pl.multiple_of(tile_base + done, 8), nr)])
            done += nr
        plsc.subcore_barrier()

        ebase = wid * (n_chunks_per_worker * CHUNK)

        def _chunk(j, _):
            off = pl.multiple_of(ebase + j * CHUNK, CHUNK)
            pltpu.sync_copy(src_hbm.at[pl.ds(off, CHUNK)], src_v)
            pltpu.sync_copy(dst_hbm.at[pl.ds(off, CHUNK)], dst_v)
            pltpu.async_copy(table_hbm.at[src_v], rows_v, sem).wait()
            pltpu.sync_copy(rows_v, acc_sh.at[dst_v], add=True)
            return 0
        lax.fori_loop(0, n_chunks_per_worker, _chunk, 0)
        plsc.subcore_barrier()

        done = 0
        while done < rows_per_tile:
            nr = min(CHUNK, rows_per_tile - done)
            lo = pl.multiple_of(tile_base + done, 8)
            pltpu.sync_copy(acc_sh.at[pl.ds(lo, nr)], rows_v.at[pl.ds(0, nr)])
            pltpu.sync_copy(rows_v.at[pl.ds(0, nr)], out_hbm.at[c, pl.ds(lo, nr)])
            done += nr

    return seg_kernel


def _deg_select(agg, idx, w_ref, b_ref):
    acc = jnp.zeros((agg.shape[0], w_ref.shape[2]), jnp.float32)
    for d in range(MAX_DEG):
        y = jnp.dot(agg, w_ref[d], preferred_element_type=jnp.float32) + b_ref[d][None, :]
        acc = jnp.where((idx == d)[:, None], y, acc)
    return acc


def _nf_layer_body(deg_col, h_ref, parts_ref, w_ref, b_ref, out_ref):
    din = h_ref.shape[1]
    agg = h_ref[...] + parts_ref[0, :, :din] + parts_ref[1, :, :din]
    degf = parts_ref[0, :, deg_col] + parts_ref[1, :, deg_col]
    idx = jnp.clip(degf.astype(jnp.int32), 1, MAX_DEG) - 1
    out_ref[...] = jnp.maximum(_deg_select(agg, idx, w_ref, b_ref), 0.0)


def _nf_layer_tc(h, parts, deg_col, w, b, blk):
    n, din = h.shape
    dout = w.shape[2]
    dw = parts.shape[2]
    grid = (n + blk - 1) // blk
    return pl.pallas_call(
        functools.partial(_nf_layer_body, deg_col),
        grid=(grid,),
        in_specs=[
            pl.BlockSpec((blk, din), lambda i: (i, 0)),
            pl.BlockSpec((2, blk, dw), lambda i: (0, i, 0)),
            pl.BlockSpec(w.shape, lambda i: (0, 0, 0)),
            pl.BlockSpec(b.shape, lambda i: (0, 0)),
        ],
        out_specs=pl.BlockSpec((blk, dout), lambda i: (i, 0)),
        out_shape=jax.ShapeDtypeStruct((n, dout), jnp.float32),
    )(h, parts, w, b)


def _readout_body(n_valid, nblocks, deg_col, h_ref, parts_ref, degp_ref, w_ref,
                  b_ref, wn_ref, bn_ref, wt_ref, bt_ref, out_ref, gsum, gmax):
    i = pl.program_id(0)
    blk = h_ref.shape[0]

    agg = h_ref[...] + parts_ref[0] + parts_ref[1]
    degf = degp_ref[0, :, deg_col] + degp_ref[1, :, deg_col]
    idx = jnp.clip(degf.astype(jnp.int32), 1, MAX_DEG) - 1
    h2 = jnp.maximum(_deg_select(agg, idx, w_ref, b_ref), 0.0)

    nf = jnp.dot(h2, wn_ref[...], preferred_element_type=jnp.float32) + bn_ref[...]
    row = i * blk + lax.broadcasted_iota(jnp.int32, (blk, 1), 0)
    valid = row < n_valid
    nsum = jnp.sum(jnp.where(valid, nf, 0.0), axis=0, keepdims=True)
    nmax = jnp.max(jnp.where(valid, nf, -jnp.inf), axis=0, keepdims=True)

    @pl.when(i == 0)
    def _():
        gsum[...] = jnp.zeros_like(gsum)
        gmax[...] = jnp.full_like(gmax, -jnp.inf)

    gsum[...] += nsum
    gmax[...] = jnp.maximum(gmax[...], nmax)

    @pl.when(i == nblocks - 1)
    def _():
        gf = jnp.maximum(jnp.concatenate([gsum[...], gmax[...]], axis=1), 0.0)
        out_ref[...] = jnp.dot(gf, wt_ref[...], preferred_element_type=jnp.float32) + bt_ref[...]


def _readout_tc(h, parts, degparts, deg_col, w, b, wn, bn, wt, bt, blk):
    n, din = h.shape
    p = wn.shape[1]
    out = wt.shape[1]
    dw = degparts.shape[2]
    grid = (n + blk - 1) // blk
    return pl.pallas_call(
        functools.partial(_readout_body, n, grid, deg_col),
        grid=(grid,),
        in_specs=[
            pl.BlockSpec((blk, din), lambda i: (i, 0)),
            pl.BlockSpec((2, blk, din), lambda i: (0, i, 0)),
            pl.BlockSpec((2, blk, dw), lambda i: (0, i, 0)),
            pl.BlockSpec(w.shape, lambda i: (0, 0, 0)),
            pl.BlockSpec(b.shape, lambda i: (0, 0)),
            pl.BlockSpec(wn.shape, lambda i: (0, 0)),
            pl.BlockSpec((1, p), lambda i: (0, 0)),
            pl.BlockSpec(wt.shape, lambda i: (0, 0)),
            pl.BlockSpec((1, out), lambda i: (0, 0)),
        ],
        out_specs=pl.BlockSpec((1, out), lambda i: (0, 0)),
        out_shape=jax.ShapeDtypeStruct((1, out), jnp.float32),
        scratch_shapes=[pltpu.VMEM((1, p), jnp.float32),
                        pltpu.VMEM((1, p), jnp.float32)],
    )(h, parts, degparts, w, b, wn, bn, wt, bt)


def kernel(x, edge_index, W1, b1, W2, b2, W_ntg, b_ntg, W_t, b_t):
    n, d = x.shape
    e = edge_index.shape[1]
    h1dim = W1.shape[2]

    d_aug = ((d + 1 + 15) // 16) * 16
    n_rows = ((n + 1 + 127) // 128) * 128
    npc = (e + NW * CHUNK - 1) // (NW * CHUNK)
    e_pad = NW * CHUNK * npc

    src = edge_index[0].astype(jnp.int32)
    dst = edge_index[1].astype(jnp.int32)
    pad = e_pad - e
    src_p = jnp.concatenate([src, jnp.zeros((pad,), jnp.int32)])
    dst_p = jnp.concatenate([dst, jnp.full((pad,), n, jnp.int32)])

    x_aug = jnp.concatenate(
        [x, jnp.ones((n, 1), jnp.float32), jnp.zeros((n, d_aug - d - 1), jnp.float32)],
        axis=1)

    seg1 = _make_segsum(n, d_aug, n_rows, npc)
    parts1 = seg1(x_aug, src_p, dst_p)

    blk = 1024
    h1 = _nf_layer_tc(x, parts1, d, W1, b1, blk)

    seg2 = _make_segsum(n, h1dim, n_rows, npc)
    parts2 = seg2(h1, src_p, dst_p)

    return _readout_tc(h1, parts2, parts1, d, W2, b2, W_ntg,
                       b_ntg.reshape(1, -1), W_t, b_t.reshape(1, -1), blk)
